# baseline (speedup 1.0000x reference)
import functools

import jax
import jax.numpy as jnp
from jax import lax
from jax.experimental import pallas as pl
from jax.experimental.pallas import tpu as pltpu
from jax.experimental.pallas import tpu_sc as plsc

H = 128
_NC = 2
_NS = 16
_B = 128
_SLICE_BYTES = 6_500_000


def _seg_sum(groups, max_slice_rows):
    max_e16 = max(E // _NS for _, lists in groups for E, _ in lists)
    plans = []
    for n_dst, lists in groups:
        half = n_dst // _NC
        P = 1
        while half // P > max_slice_rows:
            P += 1
        assert half % P == 0 and (half // P) % (16 * _NS) == 0
        plans.append((half, P, half // P, lists))

    def body(*refs):
        nl = sum(len(p[3]) for p in plans)
        outs = refs[3 * nl: 3 * nl + len(plans)]
        (chunk_src, chunk_dst, cur0, cur1, csrc0, csrc1, rows0, rows1, zbuf,
         acc, sem0, sem1) = refs[3 * nl + len(plans):]
        cid = lax.axis_index("c")
        sid = lax.axis_index("s")

        for i in range(16):
            for j in range(H // 16):
                zbuf[i, pl.ds(j * 16, 16)] = jnp.zeros((16,), jnp.float32)

        base_arg = 0
        for gi, (half, P, S, lists) in enumerate(plans):
            S16 = S // _NS
            out = outs[gi]
            for p in range(P):
                lo = cid * half + p * S
                r0 = sid * S16

                @plsc.parallel_loop(0, S16 // 16)
                def zcopy(k, r0=r0):
                    pltpu.sync_copy(zbuf, acc.at[pl.ds(r0 + k * 16, 16)])

                plsc.subcore_barrier()

                for li in range(len(lists)):
                    E, _ = lists[li]
                    e16 = E // _NS
                    nb = e16 // _B
                    table = refs[base_arg + 3 * li]
                    src = refs[base_arg + 3 * li + 1]
                    dst = refs[base_arg + 3 * li + 2]
                    off = sid * e16
                    pltpu.sync_copy(src.at[pl.ds(off, e16)],
                                    chunk_src.at[pl.ds(0, e16)])
                    pltpu.sync_copy(dst.at[pl.ds(off, e16)],
                                    chunk_dst.at[pl.ds(0, e16)])

                    @plsc.parallel_loop(0, e16 // 16, unroll=8)
                    def cbody(j, lo=lo, S=S):
                        d = chunk_dst[pl.ds(j * 16, 16)]
                        s = chunk_src[pl.ds(j * 16, 16)]
                        m = (d >= lo) & (d < lo + S)
                        scrap = S + ((lax.iota(jnp.int32, 16) * 8 + j) & 127)
                        chunk_src[pl.ds(j * 16, 16)] = jnp.where(m, s, 0)
                        chunk_dst[pl.ds(j * 16, 16)] = jnp.where(m, d - lo,
                                                                 scrap)

                    def fire(i, rbuf, sm, csrc, table=table):
                        for k in range(_B // 16):
                            csrc[pl.ds(k * 16, 16)] = (
                                chunk_src[pl.ds(i * _B + k * 16, 16)])
                        pltpu.async_copy(table.at[csrc], rbuf, sm)

                    def drain(i, rbuf, sm, cur, csrc, table=table):
                        pltpu.make_async_copy(table.at[csrc], rbuf,
                                              sm).wait()
                        for k in range(_B // 16):
                            cur[pl.ds(k * 16, 16)] = (
                                chunk_dst[pl.ds(i * _B + k * 16, 16)])
                        pltpu.sync_copy(rbuf, acc.at[cur], add=True)

                    fire(0, rows0, sem0, csrc0)
                    fire(1, rows1, sem1, csrc1)

                    def gbody(i2, carry, nb=nb):
                        i = 2 * i2
                        drain(i, rows0, sem0, cur0, csrc0)
                        fire(jnp.minimum(i + 2, nb - 2), rows0, sem0, csrc0)
                        drain(i + 1, rows1, sem1, cur1, csrc1)
                        fire(jnp.minimum(i + 3, nb - 1), rows1, sem1, csrc1)
                        return carry

                    lax.fori_loop(0, nb // 2, gbody, 0)
                    pltpu.make_async_copy(table.at[csrc0], rows0,
                                          sem0).wait()
                    pltpu.make_async_copy(table.at[csrc1], rows1,
                                          sem1).wait()

                plsc.subcore_barrier()
                g0 = cid * half + p * S + sid * S16
                pltpu.sync_copy(acc.at[pl.ds(sid * S16, S16)],
                                out.at[pl.ds(g0, S16)])
                plsc.subcore_barrier()
            base_arg += 3 * len(lists)

    mesh = plsc.VectorSubcoreMesh(core_axis_name="c", subcore_axis_name="s",
                                  num_cores=_NC, num_subcores=_NS)
    f = pl.kernel(
        body,
        out_type=tuple(
            jax.ShapeDtypeStruct((n_dst, H), jnp.float32)
            for n_dst, _ in groups),
        mesh=mesh,
        scratch_types=[
            pltpu.VMEM((max_e16,), jnp.int32),
            pltpu.VMEM((max_e16,), jnp.int32),
            pltpu.VMEM((_B,), jnp.int32),
            pltpu.VMEM((_B,), jnp.int32),
            pltpu.VMEM((_B,), jnp.int32),
            pltpu.VMEM((_B,), jnp.int32),
            pltpu.VMEM((_B, H), jnp.float32),
            pltpu.VMEM((_B, H), jnp.float32),
            pltpu.VMEM((16, H), jnp.float32),
            pltpu.VMEM_SHARED((max_slice_rows + 128, H), jnp.float32),
            pltpu.SemaphoreType.DMA,
            pltpu.SemaphoreType.DMA,
        ],
    )
    return f


def _mm_act(x, W, b, relu):
    N, K = x.shape
    M = W.shape[1]
    BN = 1024

    def body(x_ref, w_ref, b_ref, o_ref):
        y = jnp.dot(x_ref[...], w_ref[...],
                    preferred_element_type=jnp.float32) + b_ref[...]
        if relu:
            y = jnp.maximum(y, 0.0)
        o_ref[...] = y

    return pl.pallas_call(
        body,
        grid=(N // BN,),
        in_specs=[
            pl.BlockSpec((BN, K), lambda i: (i, 0)),
            pl.BlockSpec((K, M), lambda i: (0, 0)),
            pl.BlockSpec((1, M), lambda i: (0, 0)),
        ],
        out_specs=pl.BlockSpec((BN, M), lambda i: (i, 0)),
        out_shape=jax.ShapeDtypeStruct((N, M), jnp.float32),
    )(x, W, b.reshape(1, M))


def _mm_multi(x, Ws):
    N, K = x.shape
    m = Ws.shape[0]
    BN = 1024

    def body(x_ref, w_ref, *o_refs):
        xv = x_ref[...]
        for i in range(m):
            o_refs[i][...] = jnp.dot(xv, w_ref[i],
                                     preferred_element_type=jnp.float32)

    return pl.pallas_call(
        body,
        grid=(N // BN,),
        in_specs=[
            pl.BlockSpec((BN, K), lambda i: (i, 0)),
            pl.BlockSpec((m, K, H), lambda i: (0, 0, 0)),
        ],
        out_specs=[pl.BlockSpec((BN, H), lambda i: (i, 0))] * m,
        out_shape=[jax.ShapeDtypeStruct((N, H), jnp.float32)] * m,
    )(x, Ws)


def _combine_base(h, a_gt, a_gs, a_gr, d_gt, d_gs, Wroot, bsum, W1, b1, W2,
                  b2):
    N = h.shape[0]
    BN = 512

    def body(h_ref, gt_ref, gs_ref, gr_ref, dgt_ref, dgs_ref, wr_ref, bs_ref,
             w1_ref, b1_ref, w2_ref, b2_ref, o_ref):
        hv = h_ref[...]
        pre = (gt_ref[...] / jnp.maximum(dgt_ref[...], 1.0)
               + gs_ref[...] / jnp.maximum(dgs_ref[...], 1.0)
               + gr_ref[...]
               + jnp.dot(hv, wr_ref[...], preferred_element_type=jnp.float32)
               + bs_ref[...])
        h1 = jnp.maximum(
            jnp.dot(pre, w1_ref[...], preferred_element_type=jnp.float32)
            + b1_ref[...], 0.0)
        o_ref[...] = (jnp.dot(h1, w2_ref[...],
                              preferred_element_type=jnp.float32)
                      + b2_ref[...] + hv)

    row = lambda i: (i, 0)
    fixed = lambda i: (0, 0)
    return pl.pallas_call(
        body,
        grid=(N // BN,),
        in_specs=[pl.BlockSpec((BN, H), row)] * 6 + [
            pl.BlockSpec((H, H), fixed),
            pl.BlockSpec((1, H), fixed),
            pl.BlockSpec((H, H), fixed),
            pl.BlockSpec((1, H), fixed),
            pl.BlockSpec((H, H), fixed),
            pl.BlockSpec((1, H), fixed),
        ],
        out_specs=pl.BlockSpec((BN, H), row),
        out_shape=jax.ShapeDtypeStruct((N, H), jnp.float32),
    )(h, a_gt, a_gs, a_gr, d_gt, d_gs, Wroot, bsum.reshape(1, H), W1,
      b1.reshape(1, H), W2, b2.reshape(1, H))


def _combine_simple(h, a, Wroot, bsum):
    N = h.shape[0]
    BN = 1024

    def body(h_ref, a_ref, wr_ref, bs_ref, o_ref):
        hv = h_ref[...]
        o_ref[...] = jnp.maximum(
            a_ref[...]
            + jnp.dot(hv, wr_ref[...], preferred_element_type=jnp.float32)
            + bs_ref[...], 0.0) + hv

    row = lambda i: (i, 0)
    fixed = lambda i: (0, 0)
    return pl.pallas_call(
        body,
        grid=(N // BN,),
        in_specs=[
            pl.BlockSpec((BN, H), row),
            pl.BlockSpec((BN, H), row),
            pl.BlockSpec((H, H), fixed),
            pl.BlockSpec((1, H), fixed),
        ],
        out_specs=pl.BlockSpec((BN, H), row),
        out_shape=jax.ShapeDtypeStruct((N, H), jnp.float32),
    )(h, a, Wroot, bsum.reshape(1, H))


def _combine_foot_dec(h, a, Wroot, bsum, Wd, bd):
    N = h.shape[0]
    BN = 1024

    def body(h_ref, a_ref, wr_ref, bs_ref, wd_ref, bd_ref, o_ref):
        hv = h_ref[...]
        f = jnp.maximum(
            a_ref[...]
            + jnp.dot(hv, wr_ref[...], preferred_element_type=jnp.float32)
            + bs_ref[...], 0.0) + hv
        o_ref[...] = jnp.dot(f, wd_ref[...],
                             preferred_element_type=jnp.float32) + bd_ref[...]

    row = lambda i: (i, 0)
    fixed = lambda i: (0, 0)
    return pl.pallas_call(
        body,
        grid=(N // BN,),
        in_specs=[
            pl.BlockSpec((BN, H), row),
            pl.BlockSpec((BN, H), row),
            pl.BlockSpec((H, H), fixed),
            pl.BlockSpec((1, H), fixed),
            pl.BlockSpec((H, H), fixed),
            pl.BlockSpec((1, H), fixed),
        ],
        out_specs=pl.BlockSpec((BN, H), row),
        out_shape=jax.ShapeDtypeStruct((N, H), jnp.float32),
    )(h, a, Wroot, bsum.reshape(1, H), Wd, bd.reshape(1, H))


def kernel(x_base, x_joint, x_foot, params, ei_gt, ei_gs, ei_gr, ei_bj,
           ei_jj, ei_jf):
    NB, NJ, NF = x_base.shape[0], x_joint.shape[0], x_foot.shape[0]
    enc = params["enc"]

    xj = jnp.pad(x_joint, ((0, 0), (0, 384 - x_joint.shape[1])))
    Wj = jnp.pad(enc["joint"]["W"], ((0, 384 - enc["joint"]["W"].shape[0]),
                                     (0, 0)))
    xf = jnp.pad(x_foot, ((0, 0), (0, 128 - x_foot.shape[1])))
    Wf = jnp.pad(enc["foot"]["W"], ((0, 128 - enc["foot"]["W"].shape[0]),
                                    (0, 0)))

    h_b = _mm_act(x_base, enc["base"]["W"], enc["base"]["b"], True)
    h_j = _mm_act(xj, Wj, enc["joint"]["b"], True)
    h_f = _mm_act(xf, Wf, enc["foot"]["b"], True)

    src_gt, dst_gt = ei_gt[0], ei_gt[1]
    src_gs, dst_gs = ei_gs[0], ei_gs[1]
    src_gr, dst_gr = ei_gr[0], ei_gr[1]
    src_bj, dst_bj = ei_bj[0], ei_bj[1]
    src_jj, dst_jj = ei_jj[0], ei_jj[1]
    src_jf, dst_jf = ei_jf[0], ei_jf[1]
    E_bb = src_gt.shape[0]
    E_bj = src_bj.shape[0]
    E_jj = src_jj.shape[0]
    E_jf = src_jf.shape[0]

    seg_deg = _seg_sum(((NB, ((E_bb, 8),)), (NB, ((E_bb, 8),))), 2048)
    seg_layer = _seg_sum(
        ((NB, ((E_bb, NB),)), (NB, ((E_bb, NB),)), (NB, ((E_bb, NB),)),
         (NJ, ((E_bj, NB), (E_jj, NJ))), (NF, ((E_jf, NJ),))), 8192)

    ones_tab = jnp.ones((8, H), jnp.float32)
    zsrc = jnp.zeros_like(src_gt)
    d_gt, d_gs = seg_deg(ones_tab, zsrc, dst_gt, ones_tab, zsrc, dst_gs)

    bt = params["bt"]
    dec = params["dec"]
    Wd = jnp.pad(dec["W"], ((0, 0), (0, 128 - dec["W"].shape[1])))
    bd = jnp.pad(dec["b"], ((0, 128 - dec["b"].shape[0]),))

    out = None
    for l in range(3):
        lp = params["convs"][l]
        gt, gs, gr = (lp["base_gt_base"], lp["base_gs_base"],
                      lp["base_gr_base"])
        bj, jj, jf = (lp["base_connect_joint"], lp["joint_connect_joint"],
                      lp["joint_connect_foot"])

        z_gt, z_gs, z_gr, z_bj = _mm_multi(
            h_b, jnp.stack([gt["W_rel"], gs["W_rel"], gr["W_rel"],
                            bj["W_rel"]]))
        z_jj, z_jf = _mm_multi(h_j, jnp.stack([jj["W_rel"], jf["W_rel"]]))

        a_gt, a_gs, a_gr, a_j, a_f = seg_layer(
            z_gt, src_gt, dst_gt, z_gs, src_gs, dst_gs, z_gr, src_gr,
            dst_gr, z_bj, src_bj, dst_bj, z_jj, src_jj, dst_jj, z_jf,
            src_jf, dst_jf)

        wroot_b = gt["W_root"] + gs["W_root"] + gr["W_root"]
        bsum_b = gt["b_rel"] + gs["b_rel"] + gr["b_rel"]
        wroot_j = bj["W_root"] + jj["W_root"]
        bsum_j = bj["b_rel"] + jj["b_rel"]

        h_b = _combine_base(h_b, a_gt, a_gs, a_gr, d_gt, d_gs, wroot_b,
                            bsum_b, bt["W1"], bt["b1"], bt["W2"], bt["b2"])
        h_j = _combine_simple(h_j, a_j, wroot_j, bsum_j)
        if l < 2:
            h_f = _combine_simple(h_f, a_f, jf["W_root"], jf["b_rel"])
        else:
            out = _combine_foot_dec(h_f, a_f, jf["W_root"], jf["b_rel"], Wd,
                                    bd)
    return out[:, :1]


# speedup vs baseline: 47.4122x; 47.4122x over previous
import functools

import jax
import jax.numpy as jnp
from jax import lax
from jax.experimental import pallas as pl
from jax.experimental.pallas import tpu as pltpu
from jax.experimental.pallas import tpu_sc as plsc

H = 128
_NC = 2
_NS = 16
_B = 128
_SLICE_BYTES = 6_500_000


def _seg_sum(groups, max_slice_rows):
    max_e16 = max(E // _NS for _, lists in groups for E, _ in lists)
    plans = []
    for n_dst, lists in groups:
        half = n_dst // _NC
        P = 1
        while half // P > max_slice_rows:
            P += 1
        assert half % P == 0 and (half // P) % (16 * _NS) == 0
        plans.append((half, P, half // P, lists))

    def body(*refs):
        nl = sum(len(p[3]) for p in plans)
        outs = refs[3 * nl: 3 * nl + len(plans)]
        (chunk_src, chunk_dst, cur0, cur1, csrc0, csrc1, rows0, rows1, zbuf,
         acc, sem0, sem1) = refs[3 * nl + len(plans):]
        cid = lax.axis_index("c")
        sid = lax.axis_index("s")

        for i in range(16):
            for j in range(H // 16):
                zbuf[i, pl.ds(j * 16, 16)] = jnp.zeros((16,), jnp.float32)

        base_arg = 0
        for gi, (half, P, S, lists) in enumerate(plans):
            S16 = S // _NS
            out = outs[gi]
            for p in range(P):
                lo = cid * half + p * S
                r0 = sid * S16

                @plsc.parallel_loop(0, S16 // 16)
                def zcopy(k, r0=r0):
                    pltpu.sync_copy(zbuf, acc.at[pl.ds(r0 + k * 16, 16)])

                plsc.subcore_barrier()

                for li in range(len(lists)):
                    E, n_src = lists[li]
                    dmask = 1
                    while dmask * 2 <= n_src:
                        dmask *= 2
                    dmask -= 1
                    e16 = E // _NS
                    nb = e16 // _B
                    table = refs[base_arg + 3 * li]
                    src = refs[base_arg + 3 * li + 1]
                    dst = refs[base_arg + 3 * li + 2]
                    off = sid * e16
                    pltpu.sync_copy(src.at[pl.ds(off, e16)],
                                    chunk_src.at[pl.ds(0, e16)])
                    pltpu.sync_copy(dst.at[pl.ds(off, e16)],
                                    chunk_dst.at[pl.ds(0, e16)])

                    @plsc.parallel_loop(0, e16 // 16, unroll=8)
                    def cbody(j, lo=lo, S=S, dmask=dmask):
                        iota = lax.iota(jnp.int32, 16)
                        d = chunk_dst[pl.ds(j * 16, 16)]
                        s = chunk_src[pl.ds(j * 16, 16)]
                        m = (d >= lo) & (d < lo + S)
                        scrap = S + ((iota * 8 + j) & 127)
                        chunk_src[pl.ds(j * 16, 16)] = jnp.where(
                            m, s, (j * 16 + iota) & dmask)
                        chunk_dst[pl.ds(j * 16, 16)] = jnp.where(m, d - lo,
                                                                 scrap)

                    def fire(i, rbuf, sm, csrc, table=table):
                        for k in range(_B // 16):
                            csrc[pl.ds(k * 16, 16)] = (
                                chunk_src[pl.ds(i * _B + k * 16, 16)])
                        pltpu.async_copy(table.at[csrc], rbuf, sm)

                    def drain(i, rbuf, sm, cur, csrc, table=table):
                        pltpu.make_async_copy(table.at[csrc], rbuf,
                                              sm).wait()
                        for k in range(_B // 16):
                            cur[pl.ds(k * 16, 16)] = (
                                chunk_dst[pl.ds(i * _B + k * 16, 16)])
                        pltpu.sync_copy(rbuf, acc.at[cur], add=True)

                    fire(0, rows0, sem0, csrc0)
                    fire(1, rows1, sem1, csrc1)

                    def gbody(i2, carry, nb=nb):
                        i = 2 * i2
                        drain(i, rows0, sem0, cur0, csrc0)
                        fire(jnp.minimum(i + 2, nb - 2), rows0, sem0, csrc0)
                        drain(i + 1, rows1, sem1, cur1, csrc1)
                        fire(jnp.minimum(i + 3, nb - 1), rows1, sem1, csrc1)
                        return carry

                    lax.fori_loop(0, nb // 2, gbody, 0)
                    pltpu.make_async_copy(table.at[csrc0], rows0,
                                          sem0).wait()
                    pltpu.make_async_copy(table.at[csrc1], rows1,
                                          sem1).wait()

                plsc.subcore_barrier()
                g0 = cid * half + p * S + sid * S16
                pltpu.sync_copy(acc.at[pl.ds(sid * S16, S16)],
                                out.at[pl.ds(g0, S16)])
                plsc.subcore_barrier()
            base_arg += 3 * len(lists)

    mesh = plsc.VectorSubcoreMesh(core_axis_name="c", subcore_axis_name="s",
                                  num_cores=_NC, num_subcores=_NS)
    f = pl.kernel(
        body,
        out_type=tuple(
            jax.ShapeDtypeStruct((n_dst, H), jnp.float32)
            for n_dst, _ in groups),
        mesh=mesh,
        scratch_types=[
            pltpu.VMEM((max_e16,), jnp.int32),
            pltpu.VMEM((max_e16,), jnp.int32),
            pltpu.VMEM((_B,), jnp.int32),
            pltpu.VMEM((_B,), jnp.int32),
            pltpu.VMEM((_B,), jnp.int32),
            pltpu.VMEM((_B,), jnp.int32),
            pltpu.VMEM((_B, H), jnp.float32),
            pltpu.VMEM((_B, H), jnp.float32),
            pltpu.VMEM((16, H), jnp.float32),
            pltpu.VMEM_SHARED((max_slice_rows + 128, H), jnp.float32),
            pltpu.SemaphoreType.DMA,
            pltpu.SemaphoreType.DMA,
        ],
    )
    return f


def _mm_act(x, W, b, relu):
    N, K = x.shape
    M = W.shape[1]
    BN = 1024

    def body(x_ref, w_ref, b_ref, o_ref):
        y = jnp.dot(x_ref[...], w_ref[...],
                    preferred_element_type=jnp.float32) + b_ref[...]
        if relu:
            y = jnp.maximum(y, 0.0)
        o_ref[...] = y

    return pl.pallas_call(
        body,
        grid=(N // BN,),
        in_specs=[
            pl.BlockSpec((BN, K), lambda i: (i, 0)),
            pl.BlockSpec((K, M), lambda i: (0, 0)),
            pl.BlockSpec((1, M), lambda i: (0, 0)),
        ],
        out_specs=pl.BlockSpec((BN, M), lambda i: (i, 0)),
        out_shape=jax.ShapeDtypeStruct((N, M), jnp.float32),
    )(x, W, b.reshape(1, M))


def _mm_multi(x, Ws):
    N, K = x.shape
    m = Ws.shape[0]
    BN = 1024

    def body(x_ref, w_ref, *o_refs):
        xv = x_ref[...]
        for i in range(m):
            o_refs[i][...] = jnp.dot(xv, w_ref[i],
                                     preferred_element_type=jnp.float32)

    return pl.pallas_call(
        body,
        grid=(N // BN,),
        in_specs=[
            pl.BlockSpec((BN, K), lambda i: (i, 0)),
            pl.BlockSpec((m, K, H), lambda i: (0, 0, 0)),
        ],
        out_specs=[pl.BlockSpec((BN, H), lambda i: (i, 0))] * m,
        out_shape=[jax.ShapeDtypeStruct((N, H), jnp.float32)] * m,
    )(x, Ws)


def _combine_base(h, a_gt, a_gs, a_gr, d_gt, d_gs, Wroot, bsum, W1, b1, W2,
                  b2):
    N = h.shape[0]
    BN = 512

    def body(h_ref, gt_ref, gs_ref, gr_ref, dgt_ref, dgs_ref, wr_ref, bs_ref,
             w1_ref, b1_ref, w2_ref, b2_ref, o_ref):
        hv = h_ref[...]
        pre = (gt_ref[...] / jnp.maximum(dgt_ref[...], 1.0)
               + gs_ref[...] / jnp.maximum(dgs_ref[...], 1.0)
               + gr_ref[...]
               + jnp.dot(hv, wr_ref[...], preferred_element_type=jnp.float32)
               + bs_ref[...])
        h1 = jnp.maximum(
            jnp.dot(pre, w1_ref[...], preferred_element_type=jnp.float32)
            + b1_ref[...], 0.0)
        o_ref[...] = (jnp.dot(h1, w2_ref[...],
                              preferred_element_type=jnp.float32)
                      + b2_ref[...] + hv)

    row = lambda i: (i, 0)
    fixed = lambda i: (0, 0)
    return pl.pallas_call(
        body,
        grid=(N // BN,),
        in_specs=[pl.BlockSpec((BN, H), row)] * 6 + [
            pl.BlockSpec((H, H), fixed),
            pl.BlockSpec((1, H), fixed),
            pl.BlockSpec((H, H), fixed),
            pl.BlockSpec((1, H), fixed),
            pl.BlockSpec((H, H), fixed),
            pl.BlockSpec((1, H), fixed),
        ],
        out_specs=pl.BlockSpec((BN, H), row),
        out_shape=jax.ShapeDtypeStruct((N, H), jnp.float32),
    )(h, a_gt, a_gs, a_gr, d_gt, d_gs, Wroot, bsum.reshape(1, H), W1,
      b1.reshape(1, H), W2, b2.reshape(1, H))


def _combine_simple(h, a, Wroot, bsum):
    N = h.shape[0]
    BN = 1024

    def body(h_ref, a_ref, wr_ref, bs_ref, o_ref):
        hv = h_ref[...]
        o_ref[...] = jnp.maximum(
            a_ref[...]
            + jnp.dot(hv, wr_ref[...], preferred_element_type=jnp.float32)
            + bs_ref[...], 0.0) + hv

    row = lambda i: (i, 0)
    fixed = lambda i: (0, 0)
    return pl.pallas_call(
        body,
        grid=(N // BN,),
        in_specs=[
            pl.BlockSpec((BN, H), row),
            pl.BlockSpec((BN, H), row),
            pl.BlockSpec((H, H), fixed),
            pl.BlockSpec((1, H), fixed),
        ],
        out_specs=pl.BlockSpec((BN, H), row),
        out_shape=jax.ShapeDtypeStruct((N, H), jnp.float32),
    )(h, a, Wroot, bsum.reshape(1, H))


def _combine_foot_dec(h, a, Wroot, bsum, Wd, bd):
    N = h.shape[0]
    BN = 1024

    def body(h_ref, a_ref, wr_ref, bs_ref, wd_ref, bd_ref, o_ref):
        hv = h_ref[...]
        f = jnp.maximum(
            a_ref[...]
            + jnp.dot(hv, wr_ref[...], preferred_element_type=jnp.float32)
            + bs_ref[...], 0.0) + hv
        o_ref[...] = jnp.dot(f, wd_ref[...],
                             preferred_element_type=jnp.float32) + bd_ref[...]

    row = lambda i: (i, 0)
    fixed = lambda i: (0, 0)
    return pl.pallas_call(
        body,
        grid=(N // BN,),
        in_specs=[
            pl.BlockSpec((BN, H), row),
            pl.BlockSpec((BN, H), row),
            pl.BlockSpec((H, H), fixed),
            pl.BlockSpec((1, H), fixed),
            pl.BlockSpec((H, H), fixed),
            pl.BlockSpec((1, H), fixed),
        ],
        out_specs=pl.BlockSpec((BN, H), row),
        out_shape=jax.ShapeDtypeStruct((N, H), jnp.float32),
    )(h, a, Wroot, bsum.reshape(1, H), Wd, bd.reshape(1, H))


def kernel(x_base, x_joint, x_foot, params, ei_gt, ei_gs, ei_gr, ei_bj,
           ei_jj, ei_jf):
    NB, NJ, NF = x_base.shape[0], x_joint.shape[0], x_foot.shape[0]
    enc = params["enc"]

    xj = jnp.pad(x_joint, ((0, 0), (0, 384 - x_joint.shape[1])))
    Wj = jnp.pad(enc["joint"]["W"], ((0, 384 - enc["joint"]["W"].shape[0]),
                                     (0, 0)))
    xf = jnp.pad(x_foot, ((0, 0), (0, 128 - x_foot.shape[1])))
    Wf = jnp.pad(enc["foot"]["W"], ((0, 128 - enc["foot"]["W"].shape[0]),
                                    (0, 0)))

    h_b = _mm_act(x_base, enc["base"]["W"], enc["base"]["b"], True)
    h_j = _mm_act(xj, Wj, enc["joint"]["b"], True)
    h_f = _mm_act(xf, Wf, enc["foot"]["b"], True)

    src_gt, dst_gt = ei_gt[0], ei_gt[1]
    src_gs, dst_gs = ei_gs[0], ei_gs[1]
    src_gr, dst_gr = ei_gr[0], ei_gr[1]
    src_bj, dst_bj = ei_bj[0], ei_bj[1]
    src_jj, dst_jj = ei_jj[0], ei_jj[1]
    src_jf, dst_jf = ei_jf[0], ei_jf[1]
    E_bb = src_gt.shape[0]
    E_bj = src_bj.shape[0]
    E_jj = src_jj.shape[0]
    E_jf = src_jf.shape[0]

    seg_deg = _seg_sum(((NB, ((E_bb, 4096),)), (NB, ((E_bb, 4096),))), 2048)
    seg_layer = _seg_sum(
        ((NB, ((E_bb, NB),)), (NB, ((E_bb, NB),)), (NB, ((E_bb, NB),)),
         (NJ, ((E_bj, NB), (E_jj, NJ))), (NF, ((E_jf, NJ),))), 8192)

    ones_tab = jnp.ones((4096, H), jnp.float32)
    zsrc = jnp.zeros_like(src_gt)
    d_gt, d_gs = seg_deg(ones_tab, zsrc, dst_gt, ones_tab, zsrc, dst_gs)

    bt = params["bt"]
    dec = params["dec"]
    Wd = jnp.pad(dec["W"], ((0, 0), (0, 128 - dec["W"].shape[1])))
    bd = jnp.pad(dec["b"], ((0, 128 - dec["b"].shape[0]),))

    out = None
    for l in range(3):
        lp = params["convs"][l]
        gt, gs, gr = (lp["base_gt_base"], lp["base_gs_base"],
                      lp["base_gr_base"])
        bj, jj, jf = (lp["base_connect_joint"], lp["joint_connect_joint"],
                      lp["joint_connect_foot"])

        z_gt, z_gs, z_gr, z_bj = _mm_multi(
            h_b, jnp.stack([gt["W_rel"], gs["W_rel"], gr["W_rel"],
                            bj["W_rel"]]))
        z_jj, z_jf = _mm_multi(h_j, jnp.stack([jj["W_rel"], jf["W_rel"]]))

        a_gt, a_gs, a_gr, a_j, a_f = seg_layer(
            z_gt, src_gt, dst_gt, z_gs, src_gs, dst_gs, z_gr, src_gr,
            dst_gr, z_bj, src_bj, dst_bj, z_jj, src_jj, dst_jj, z_jf,
            src_jf, dst_jf)

        wroot_b = gt["W_root"] + gs["W_root"] + gr["W_root"]
        bsum_b = gt["b_rel"] + gs["b_rel"] + gr["b_rel"]
        wroot_j = bj["W_root"] + jj["W_root"]
        bsum_j = bj["b_rel"] + jj["b_rel"]

        h_b = _combine_base(h_b, a_gt, a_gs, a_gr, d_gt, d_gs, wroot_b,
                            bsum_b, bt["W1"], bt["b1"], bt["W2"], bt["b2"])
        h_j = _combine_simple(h_j, a_j, wroot_j, bsum_j)
        if l < 2:
            h_f = _combine_simple(h_f, a_f, jf["W_root"], jf["b_rel"])
        else:
            out = _combine_foot_dec(h_f, a_f, jf["W_root"], jf["b_rel"], Wd,
                                    bd)
    return out[:, :1]


# speedup vs baseline: 48.3064x; 1.0189x over previous
import functools

import jax
import jax.numpy as jnp
from jax import lax
from jax.experimental import pallas as pl
from jax.experimental.pallas import tpu as pltpu
from jax.experimental.pallas import tpu_sc as plsc

H = 128
_NC = 2
_NS = 16
_B = 128
_SLICE_BYTES = 6_500_000


def _seg_sum(groups, max_slice_rows):
    max_e16 = max(E // _NS for _, lists in groups for E, _ in lists)
    plans = []
    for n_dst, lists in groups:
        half = n_dst // _NC
        P = 1
        while half // P > max_slice_rows:
            P += 1
        assert half % P == 0 and (half // P) % (16 * _NS) == 0
        plans.append((half, P, half // P, lists))

    def body(*refs):
        nl = sum(len(p[3]) for p in plans)
        outs = refs[3 * nl: 3 * nl + len(plans)]
        (chunk_src, chunk_dst, cur0, cur1, csrc0, csrc1, rows0, rows1, zbuf,
         acc, sem0, sem1) = refs[3 * nl + len(plans):]
        cid = lax.axis_index("c")
        sid = lax.axis_index("s")

        for i in range(16):
            for j in range(H // 16):
                zbuf[i, pl.ds(j * 16, 16)] = jnp.zeros((16,), jnp.float32)

        base_arg = 0
        for gi, (half, P, S, lists) in enumerate(plans):
            S16 = S // _NS
            out = outs[gi]
            for p in range(P):
                lo = cid * half + p * S
                r0 = sid * S16

                @plsc.parallel_loop(0, S16 // 16)
                def zcopy(k, r0=r0):
                    pltpu.sync_copy(zbuf, acc.at[pl.ds(r0 + k * 16, 16)])

                plsc.subcore_barrier()

                for li in range(len(lists)):
                    E, n_src = lists[li]
                    dmask = 1
                    while dmask * 2 <= n_src:
                        dmask *= 2
                    dmask -= 1
                    e16 = E // _NS
                    nb = e16 // _B
                    table = refs[base_arg + 3 * li]
                    src = refs[base_arg + 3 * li + 1]
                    dst = refs[base_arg + 3 * li + 2]
                    off = sid * e16
                    pltpu.sync_copy(src.at[pl.ds(off, e16)],
                                    chunk_src.at[pl.ds(0, e16)])
                    pltpu.sync_copy(dst.at[pl.ds(off, e16)],
                                    chunk_dst.at[pl.ds(0, e16)])

                    @plsc.parallel_loop(0, e16 // 16, unroll=8)
                    def cbody(j, lo=lo, S=S, dmask=dmask):
                        iota = lax.iota(jnp.int32, 16)
                        d = chunk_dst[pl.ds(j * 16, 16)]
                        s = chunk_src[pl.ds(j * 16, 16)]
                        m = (d >= lo) & (d < lo + S)
                        scrap = S + ((iota * 8 + j) & 127)
                        chunk_src[pl.ds(j * 16, 16)] = jnp.where(
                            m, s, (j * 16 + iota) & dmask)
                        chunk_dst[pl.ds(j * 16, 16)] = jnp.where(m, d - lo,
                                                                 scrap)

                    def fire(i, rbuf, sm, csrc, table=table):
                        for k in range(_B // 16):
                            csrc[pl.ds(k * 16, 16)] = (
                                chunk_src[pl.ds(i * _B + k * 16, 16)])
                        pltpu.async_copy(table.at[csrc], rbuf, sm)

                    def drain(i, rbuf, sm, cur, csrc, table=table):
                        pltpu.make_async_copy(table.at[csrc], rbuf,
                                              sm).wait()
                        for k in range(_B // 16):
                            cur[pl.ds(k * 16, 16)] = (
                                chunk_dst[pl.ds(i * _B + k * 16, 16)])
                        pltpu.sync_copy(rbuf, acc.at[cur], add=True)

                    fire(0, rows0, sem0, csrc0)
                    fire(1, rows1, sem1, csrc1)

                    def gbody(i2, carry, nb=nb):
                        i = 2 * i2
                        drain(i, rows0, sem0, cur0, csrc0)
                        fire(jnp.minimum(i + 2, nb - 2), rows0, sem0, csrc0)
                        drain(i + 1, rows1, sem1, cur1, csrc1)
                        fire(jnp.minimum(i + 3, nb - 1), rows1, sem1, csrc1)
                        return carry

                    lax.fori_loop(0, nb // 2, gbody, 0)
                    pltpu.make_async_copy(table.at[csrc0], rows0,
                                          sem0).wait()
                    pltpu.make_async_copy(table.at[csrc1], rows1,
                                          sem1).wait()

                plsc.subcore_barrier()
                g0 = cid * half + p * S + sid * S16
                pltpu.sync_copy(acc.at[pl.ds(sid * S16, S16)],
                                out.at[pl.ds(g0, S16)])
                plsc.subcore_barrier()
            base_arg += 3 * len(lists)

    mesh = plsc.VectorSubcoreMesh(core_axis_name="c", subcore_axis_name="s",
                                  num_cores=_NC, num_subcores=_NS)
    f = pl.kernel(
        body,
        out_type=tuple(
            jax.ShapeDtypeStruct((n_dst, H), jnp.float32)
            for n_dst, _ in groups),
        mesh=mesh,
        scratch_types=[
            pltpu.VMEM((max_e16,), jnp.int32),
            pltpu.VMEM((max_e16,), jnp.int32),
            pltpu.VMEM((_B,), jnp.int32),
            pltpu.VMEM((_B,), jnp.int32),
            pltpu.VMEM((_B,), jnp.int32),
            pltpu.VMEM((_B,), jnp.int32),
            pltpu.VMEM((_B, H), jnp.float32),
            pltpu.VMEM((_B, H), jnp.float32),
            pltpu.VMEM((16, H), jnp.float32),
            pltpu.VMEM_SHARED((max_slice_rows + 128, H), jnp.float32),
            pltpu.SemaphoreType.DMA,
            pltpu.SemaphoreType.DMA,
        ],
    )
    return f


def _mm_act(x, W, b, relu):
    N, K = x.shape
    M = W.shape[1]
    BN = 1024

    def body(x_ref, w_ref, b_ref, o_ref):
        y = jnp.dot(x_ref[...], w_ref[...],
                    preferred_element_type=jnp.float32,
                    precision=lax.Precision.HIGHEST) + b_ref[...]
        if relu:
            y = jnp.maximum(y, 0.0)
        o_ref[...] = y

    return pl.pallas_call(
        body,
        grid=(N // BN,),
        in_specs=[
            pl.BlockSpec((BN, K), lambda i: (i, 0)),
            pl.BlockSpec((K, M), lambda i: (0, 0)),
            pl.BlockSpec((1, M), lambda i: (0, 0)),
        ],
        out_specs=pl.BlockSpec((BN, M), lambda i: (i, 0)),
        out_shape=jax.ShapeDtypeStruct((N, M), jnp.float32),
    )(x, W, b.reshape(1, M))


def _mm_multi(x, Ws):
    N, K = x.shape
    m = Ws.shape[0]
    BN = 1024

    def body(x_ref, w_ref, *o_refs):
        xv = x_ref[...]
        for i in range(m):
            o_refs[i][...] = jnp.dot(xv, w_ref[i],
                                     preferred_element_type=jnp.float32,
                    precision=lax.Precision.HIGHEST)

    return pl.pallas_call(
        body,
        grid=(N // BN,),
        in_specs=[
            pl.BlockSpec((BN, K), lambda i: (i, 0)),
            pl.BlockSpec((m, K, H), lambda i: (0, 0, 0)),
        ],
        out_specs=[pl.BlockSpec((BN, H), lambda i: (i, 0))] * m,
        out_shape=[jax.ShapeDtypeStruct((N, H), jnp.float32)] * m,
    )(x, Ws)


def _combine_base(h, a_gt, a_gs, a_gr, d_gt, d_gs, Wroot, bsum, W1, b1, W2,
                  b2):
    N = h.shape[0]
    BN = 512

    def body(h_ref, gt_ref, gs_ref, gr_ref, dgt_ref, dgs_ref, wr_ref, bs_ref,
             w1_ref, b1_ref, w2_ref, b2_ref, o_ref):
        hv = h_ref[...]
        pre = (gt_ref[...] / jnp.maximum(dgt_ref[...], 1.0)
               + gs_ref[...] / jnp.maximum(dgs_ref[...], 1.0)
               + gr_ref[...]
               + jnp.dot(hv, wr_ref[...], preferred_element_type=jnp.float32,
                    precision=lax.Precision.HIGHEST)
               + bs_ref[...])
        h1 = jnp.maximum(
            jnp.dot(pre, w1_ref[...], preferred_element_type=jnp.float32,
                    precision=lax.Precision.HIGHEST)
            + b1_ref[...], 0.0)
        o_ref[...] = (jnp.dot(h1, w2_ref[...],
                              preferred_element_type=jnp.float32,
                    precision=lax.Precision.HIGHEST)
                      + b2_ref[...] + hv)

    row = lambda i: (i, 0)
    fixed = lambda i: (0, 0)
    return pl.pallas_call(
        body,
        grid=(N // BN,),
        in_specs=[pl.BlockSpec((BN, H), row)] * 6 + [
            pl.BlockSpec((H, H), fixed),
            pl.BlockSpec((1, H), fixed),
            pl.BlockSpec((H, H), fixed),
            pl.BlockSpec((1, H), fixed),
            pl.BlockSpec((H, H), fixed),
            pl.BlockSpec((1, H), fixed),
        ],
        out_specs=pl.BlockSpec((BN, H), row),
        out_shape=jax.ShapeDtypeStruct((N, H), jnp.float32),
    )(h, a_gt, a_gs, a_gr, d_gt, d_gs, Wroot, bsum.reshape(1, H), W1,
      b1.reshape(1, H), W2, b2.reshape(1, H))


def _combine_simple(h, a, Wroot, bsum):
    N = h.shape[0]
    BN = 1024

    def body(h_ref, a_ref, wr_ref, bs_ref, o_ref):
        hv = h_ref[...]
        o_ref[...] = jnp.maximum(
            a_ref[...]
            + jnp.dot(hv, wr_ref[...], preferred_element_type=jnp.float32,
                    precision=lax.Precision.HIGHEST)
            + bs_ref[...], 0.0) + hv

    row = lambda i: (i, 0)
    fixed = lambda i: (0, 0)
    return pl.pallas_call(
        body,
        grid=(N // BN,),
        in_specs=[
            pl.BlockSpec((BN, H), row),
            pl.BlockSpec((BN, H), row),
            pl.BlockSpec((H, H), fixed),
            pl.BlockSpec((1, H), fixed),
        ],
        out_specs=pl.BlockSpec((BN, H), row),
        out_shape=jax.ShapeDtypeStruct((N, H), jnp.float32),
    )(h, a, Wroot, bsum.reshape(1, H))


def _combine_foot_dec(h, a, Wroot, bsum, Wd, bd):
    N = h.shape[0]
    BN = 1024

    def body(h_ref, a_ref, wr_ref, bs_ref, wd_ref, bd_ref, o_ref):
        hv = h_ref[...]
        f = jnp.maximum(
            a_ref[...]
            + jnp.dot(hv, wr_ref[...], preferred_element_type=jnp.float32,
                    precision=lax.Precision.HIGHEST)
            + bs_ref[...], 0.0) + hv
        o_ref[...] = jnp.dot(f, wd_ref[...],
                             preferred_element_type=jnp.float32,
                    precision=lax.Precision.HIGHEST) + bd_ref[...]

    row = lambda i: (i, 0)
    fixed = lambda i: (0, 0)
    return pl.pallas_call(
        body,
        grid=(N // BN,),
        in_specs=[
            pl.BlockSpec((BN, H), row),
            pl.BlockSpec((BN, H), row),
            pl.BlockSpec((H, H), fixed),
            pl.BlockSpec((1, H), fixed),
            pl.BlockSpec((H, H), fixed),
            pl.BlockSpec((1, H), fixed),
        ],
        out_specs=pl.BlockSpec((BN, H), row),
        out_shape=jax.ShapeDtypeStruct((N, H), jnp.float32),
    )(h, a, Wroot, bsum.reshape(1, H), Wd, bd.reshape(1, H))


def kernel(x_base, x_joint, x_foot, params, ei_gt, ei_gs, ei_gr, ei_bj,
           ei_jj, ei_jf):
    NB, NJ, NF = x_base.shape[0], x_joint.shape[0], x_foot.shape[0]
    enc = params["enc"]

    xj = jnp.pad(x_joint, ((0, 0), (0, 384 - x_joint.shape[1])))
    Wj = jnp.pad(enc["joint"]["W"], ((0, 384 - enc["joint"]["W"].shape[0]),
                                     (0, 0)))
    xf = jnp.pad(x_foot, ((0, 0), (0, 128 - x_foot.shape[1])))
    Wf = jnp.pad(enc["foot"]["W"], ((0, 128 - enc["foot"]["W"].shape[0]),
                                    (0, 0)))

    h_b = _mm_act(x_base, enc["base"]["W"], enc["base"]["b"], True)
    h_j = _mm_act(xj, Wj, enc["joint"]["b"], True)
    h_f = _mm_act(xf, Wf, enc["foot"]["b"], True)

    src_gt, dst_gt = ei_gt[0], ei_gt[1]
    src_gs, dst_gs = ei_gs[0], ei_gs[1]
    src_gr, dst_gr = ei_gr[0], ei_gr[1]
    src_bj, dst_bj = ei_bj[0], ei_bj[1]
    src_jj, dst_jj = ei_jj[0], ei_jj[1]
    src_jf, dst_jf = ei_jf[0], ei_jf[1]
    E_bb = src_gt.shape[0]
    E_bj = src_bj.shape[0]
    E_jj = src_jj.shape[0]
    E_jf = src_jf.shape[0]

    seg_deg = _seg_sum(((NB, ((E_bb, 4096),)), (NB, ((E_bb, 4096),))), 8192)
    seg_layer = _seg_sum(
        ((NB, ((E_bb, NB),)), (NB, ((E_bb, NB),)), (NB, ((E_bb, NB),)),
         (NJ, ((E_bj, NB), (E_jj, NJ))), (NF, ((E_jf, NJ),))), 8192)

    ones_tab = jnp.ones((4096, H), jnp.float32)
    zsrc = jnp.zeros_like(src_gt)
    d_gt, d_gs = seg_deg(ones_tab, zsrc, dst_gt, ones_tab, zsrc, dst_gs)

    bt = params["bt"]
    dec = params["dec"]
    Wd = jnp.pad(dec["W"], ((0, 0), (0, 128 - dec["W"].shape[1])))
    bd = jnp.pad(dec["b"], ((0, 128 - dec["b"].shape[0]),))

    out = None
    for l in range(3):
        lp = params["convs"][l]
        gt, gs, gr = (lp["base_gt_base"], lp["base_gs_base"],
                      lp["base_gr_base"])
        bj, jj, jf = (lp["base_connect_joint"], lp["joint_connect_joint"],
                      lp["joint_connect_foot"])

        z_gt, z_gs, z_gr, z_bj = _mm_multi(
            h_b, jnp.stack([gt["W_rel"], gs["W_rel"], gr["W_rel"],
                            bj["W_rel"]]))
        z_jj, z_jf = _mm_multi(h_j, jnp.stack([jj["W_rel"], jf["W_rel"]]))

        a_gt, a_gs, a_gr, a_j, a_f = seg_layer(
            z_gt, src_gt, dst_gt, z_gs, src_gs, dst_gs, z_gr, src_gr,
            dst_gr, z_bj, src_bj, dst_bj, z_jj, src_jj, dst_jj, z_jf,
            src_jf, dst_jf)

        wroot_b = gt["W_root"] + gs["W_root"] + gr["W_root"]
        bsum_b = gt["b_rel"] + gs["b_rel"] + gr["b_rel"]
        wroot_j = bj["W_root"] + jj["W_root"]
        bsum_j = bj["b_rel"] + jj["b_rel"]

        h_b = _combine_base(h_b, a_gt, a_gs, a_gr, d_gt, d_gs, wroot_b,
                            bsum_b, bt["W1"], bt["b1"], bt["W2"], bt["b2"])
        h_j = _combine_simple(h_j, a_j, wroot_j, bsum_j)
        if l < 2:
            h_f = _combine_simple(h_f, a_f, jf["W_root"], jf["b_rel"])
        else:
            out = _combine_foot_dec(h_f, a_f, jf["W_root"], jf["b_rel"], Wd,
                                    bd)
    return out[:, :1]


# speedup vs baseline: 49.7680x; 1.0303x over previous
import functools

import jax
import jax.numpy as jnp
from jax import lax
from jax.experimental import pallas as pl
from jax.experimental.pallas import tpu as pltpu
from jax.experimental.pallas import tpu_sc as plsc

H = 128
_NC = 2
_NS = 16
_B = 128
_SLICE_BYTES = 6_500_000


def _seg_sum(groups, max_slice_rows):
    max_e16 = max(E // _NS for _, lists in groups for E, _ in lists)
    plans = []
    for n_dst, lists in groups:
        half = n_dst // _NC
        P = 1
        while half // P > max_slice_rows:
            P += 1
        assert half % P == 0 and (half // P) % (16 * _NS) == 0
        plans.append((half, P, half // P, lists))

    def body(*refs):
        nl = sum(len(p[3]) for p in plans)
        outs = refs[3 * nl: 3 * nl + len(plans)]
        (chunk_src, chunk_dst, cur0, cur1, csrc0, csrc1, rows0, rows1, zbuf,
         acc, sem0, sem1) = refs[3 * nl + len(plans):]
        cid = lax.axis_index("c")
        sid = lax.axis_index("s")

        for i in range(16):
            for j in range(H // 16):
                zbuf[i, pl.ds(j * 16, 16)] = jnp.zeros((16,), jnp.float32)

        base_arg = 0
        for gi, (half, P, S, lists) in enumerate(plans):
            S16 = S // _NS
            out = outs[gi]
            for p in range(P):
                lo = cid * half + p * S
                r0 = sid * S16

                @plsc.parallel_loop(0, S16 // 16)
                def zcopy(k, r0=r0):
                    pltpu.sync_copy(zbuf, acc.at[pl.ds(r0 + k * 16, 16)])

                plsc.subcore_barrier()

                for li in range(len(lists)):
                    E, n_src = lists[li]
                    dmask = 1
                    while dmask * 2 <= n_src:
                        dmask *= 2
                    dmask -= 1
                    e16 = E // _NS
                    nb = e16 // _B
                    table = refs[base_arg + 3 * li]
                    src = refs[base_arg + 3 * li + 1]
                    dst = refs[base_arg + 3 * li + 2]
                    off = sid * e16
                    pltpu.sync_copy(src.at[pl.ds(off, e16)],
                                    chunk_src.at[pl.ds(0, e16)])
                    pltpu.sync_copy(dst.at[pl.ds(off, e16)],
                                    chunk_dst.at[pl.ds(0, e16)])

                    @plsc.parallel_loop(0, e16 // 16, unroll=8)
                    def cbody(j, lo=lo, S=S, dmask=dmask):
                        iota = lax.iota(jnp.int32, 16)
                        d = chunk_dst[pl.ds(j * 16, 16)]
                        s = chunk_src[pl.ds(j * 16, 16)]
                        m = (d >= lo) & (d < lo + S)
                        scrap = S + ((iota * 8 + j) & 127)
                        chunk_src[pl.ds(j * 16, 16)] = jnp.where(
                            m, s, (j * 16 + iota) & dmask)
                        chunk_dst[pl.ds(j * 16, 16)] = jnp.where(m, d - lo,
                                                                 scrap)

                    def fire(i, rbuf, sm, csrc, table=table):
                        for k in range(_B // 16):
                            csrc[pl.ds(k * 16, 16)] = (
                                chunk_src[pl.ds(i * _B + k * 16, 16)])
                        pltpu.async_copy(table.at[csrc], rbuf, sm)

                    def drain(i, rbuf, sm, cur, csrc, table=table):
                        pltpu.make_async_copy(table.at[csrc], rbuf,
                                              sm).wait()
                        for k in range(_B // 16):
                            cur[pl.ds(k * 16, 16)] = (
                                chunk_dst[pl.ds(i * _B + k * 16, 16)])
                        pltpu.sync_copy(rbuf, acc.at[cur], add=True)

                    fire(0, rows0, sem0, csrc0)
                    fire(1, rows1, sem1, csrc1)

                    def gbody(i2, carry, nb=nb):
                        i = 2 * i2
                        drain(i, rows0, sem0, cur0, csrc0)
                        fire(jnp.minimum(i + 2, nb - 2), rows0, sem0, csrc0)
                        drain(i + 1, rows1, sem1, cur1, csrc1)
                        fire(jnp.minimum(i + 3, nb - 1), rows1, sem1, csrc1)
                        return carry

                    lax.fori_loop(0, nb // 2, gbody, 0)
                    pltpu.make_async_copy(table.at[csrc0], rows0,
                                          sem0).wait()
                    pltpu.make_async_copy(table.at[csrc1], rows1,
                                          sem1).wait()

                plsc.subcore_barrier()
                g0 = cid * half + p * S + sid * S16
                pltpu.sync_copy(acc.at[pl.ds(sid * S16, S16)],
                                out.at[pl.ds(g0, S16)])
                plsc.subcore_barrier()
            base_arg += 3 * len(lists)

    mesh = plsc.VectorSubcoreMesh(core_axis_name="c", subcore_axis_name="s",
                                  num_cores=_NC, num_subcores=_NS)
    f = pl.kernel(
        body,
        out_type=tuple(
            jax.ShapeDtypeStruct((n_dst, H), jnp.float32)
            for n_dst, _ in groups),
        mesh=mesh,
        scratch_types=[
            pltpu.VMEM((max_e16,), jnp.int32),
            pltpu.VMEM((max_e16,), jnp.int32),
            pltpu.VMEM((_B,), jnp.int32),
            pltpu.VMEM((_B,), jnp.int32),
            pltpu.VMEM((_B,), jnp.int32),
            pltpu.VMEM((_B,), jnp.int32),
            pltpu.VMEM((_B, H), jnp.float32),
            pltpu.VMEM((_B, H), jnp.float32),
            pltpu.VMEM((16, H), jnp.float32),
            pltpu.VMEM_SHARED((max_slice_rows + 128, H), jnp.float32),
            pltpu.SemaphoreType.DMA,
            pltpu.SemaphoreType.DMA,
        ],
    )
    return f


def _mm_act(x, W, b, relu):
    N, K = x.shape
    M = W.shape[1]
    BN = 1024

    def body(x_ref, w_ref, b_ref, o_ref):
        y = jnp.dot(x_ref[...], w_ref[...],
                    preferred_element_type=jnp.float32) + b_ref[...]
        if relu:
            y = jnp.maximum(y, 0.0)
        o_ref[...] = y

    return pl.pallas_call(
        body,
        grid=(N // BN,),
        in_specs=[
            pl.BlockSpec((BN, K), lambda i: (i, 0)),
            pl.BlockSpec((K, M), lambda i: (0, 0)),
            pl.BlockSpec((1, M), lambda i: (0, 0)),
        ],
        out_specs=pl.BlockSpec((BN, M), lambda i: (i, 0)),
        out_shape=jax.ShapeDtypeStruct((N, M), jnp.float32),
    )(x, W, b.reshape(1, M))


def _mm_multi(x, Ws):
    N, K = x.shape
    m = Ws.shape[0]
    BN = 1024

    def body(x_ref, w_ref, *o_refs):
        xv = x_ref[...]
        for i in range(m):
            o_refs[i][...] = jnp.dot(xv, w_ref[i],
                                     preferred_element_type=jnp.float32)

    return pl.pallas_call(
        body,
        grid=(N // BN,),
        in_specs=[
            pl.BlockSpec((BN, K), lambda i: (i, 0)),
            pl.BlockSpec((m, K, H), lambda i: (0, 0, 0)),
        ],
        out_specs=[pl.BlockSpec((BN, H), lambda i: (i, 0))] * m,
        out_shape=[jax.ShapeDtypeStruct((N, H), jnp.float32)] * m,
    )(x, Ws)


def _combine_base(h, a_gt, a_gs, a_gr, d_gt, d_gs, W_gt, W_gs, W_gr, Wroot,
                  bsum, W1, b1, W2, b2):
    N = h.shape[0]
    BN = 512

    def body(h_ref, gt_ref, gs_ref, gr_ref, dgt_ref, dgs_ref, wgt_ref,
             wgs_ref, wgr_ref, wr_ref, bs_ref, w1_ref, b1_ref, w2_ref,
             b2_ref, o_ref):
        hv = h_ref[...]
        dot = lambda a, b: jnp.dot(a, b, preferred_element_type=jnp.float32)
        pre = (dot(gt_ref[...] / jnp.maximum(dgt_ref[...], 1.0), wgt_ref[...])
               + dot(gs_ref[...] / jnp.maximum(dgs_ref[...], 1.0),
                     wgs_ref[...])
               + dot(gr_ref[...], wgr_ref[...])
               + dot(hv, wr_ref[...]) + bs_ref[...])
        h1 = jnp.maximum(dot(pre, w1_ref[...]) + b1_ref[...], 0.0)
        o_ref[...] = dot(h1, w2_ref[...]) + b2_ref[...] + hv

    row = lambda i: (i, 0)
    fixed = lambda i: (0, 0)
    wspec = [pl.BlockSpec((H, H), fixed)]
    bspec = [pl.BlockSpec((1, H), fixed)]
    return pl.pallas_call(
        body,
        grid=(N // BN,),
        in_specs=[pl.BlockSpec((BN, H), row)] * 6 + wspec * 4 + bspec
        + wspec + bspec + wspec + bspec,
        out_specs=pl.BlockSpec((BN, H), row),
        out_shape=jax.ShapeDtypeStruct((N, H), jnp.float32),
    )(h, a_gt, a_gs, a_gr, d_gt, d_gs, W_gt, W_gs, W_gr, Wroot,
      bsum.reshape(1, H), W1, b1.reshape(1, H), W2, b2.reshape(1, H))


def _combine_two(h, a1, a2, W1r, W2r, Wroot, bsum):
    N = h.shape[0]
    BN = 1024
    two = a2 is not None

    def body(*refs):
        if two:
            h_ref, a1_ref, a2_ref, w1_ref, w2_ref, wr_ref, bs_ref, o_ref = (
                refs)
        else:
            h_ref, a1_ref, w1_ref, wr_ref, bs_ref, o_ref = refs
        hv = h_ref[...]
        dot = lambda a, b: jnp.dot(a, b, preferred_element_type=jnp.float32)
        pre = dot(a1_ref[...], w1_ref[...])
        if two:
            pre = pre + dot(a2_ref[...], w2_ref[...])
        o_ref[...] = jnp.maximum(
            pre + dot(hv, wr_ref[...]) + bs_ref[...], 0.0) + hv

    row = lambda i: (i, 0)
    fixed = lambda i: (0, 0)
    ins = [h, a1] + ([a2] if two else []) + [W1r] + ([W2r] if two else []) \
        + [Wroot, bsum.reshape(1, H)]
    specs = ([pl.BlockSpec((BN, H), row)] * (3 if two else 2)
             + [pl.BlockSpec((H, H), fixed)] * (3 if two else 2)
             + [pl.BlockSpec((1, H), fixed)])
    return pl.pallas_call(
        body,
        grid=(N // BN,),
        in_specs=specs,
        out_specs=pl.BlockSpec((BN, H), row),
        out_shape=jax.ShapeDtypeStruct((N, H), jnp.float32),
    )(*ins)


def _combine_foot_dec(h, a, Wrel, Wroot, bsum, Wd, bd):
    N = h.shape[0]
    BN = 1024

    def body(h_ref, a_ref, wl_ref, wr_ref, bs_ref, wd_ref, bd_ref, o_ref):
        hv = h_ref[...]
        dot = lambda a, b: jnp.dot(a, b, preferred_element_type=jnp.float32)
        f = jnp.maximum(
            dot(a_ref[...], wl_ref[...])
            + dot(hv, wr_ref[...]) + bs_ref[...], 0.0) + hv
        o_ref[...] = dot(f, wd_ref[...]) + bd_ref[...]

    row = lambda i: (i, 0)
    fixed = lambda i: (0, 0)
    return pl.pallas_call(
        body,
        grid=(N // BN,),
        in_specs=[
            pl.BlockSpec((BN, H), row),
            pl.BlockSpec((BN, H), row),
            pl.BlockSpec((H, H), fixed),
            pl.BlockSpec((H, H), fixed),
            pl.BlockSpec((1, H), fixed),
            pl.BlockSpec((H, H), fixed),
            pl.BlockSpec((1, H), fixed),
        ],
        out_specs=pl.BlockSpec((BN, H), row),
        out_shape=jax.ShapeDtypeStruct((N, H), jnp.float32),
    )(h, a, Wrel, Wroot, bsum.reshape(1, H), Wd, bd.reshape(1, H))


def kernel(x_base, x_joint, x_foot, params, ei_gt, ei_gs, ei_gr, ei_bj,
           ei_jj, ei_jf):
    NB, NJ, NF = x_base.shape[0], x_joint.shape[0], x_foot.shape[0]
    enc = params["enc"]

    xj = jnp.pad(x_joint, ((0, 0), (0, 384 - x_joint.shape[1])))
    Wj = jnp.pad(enc["joint"]["W"], ((0, 384 - enc["joint"]["W"].shape[0]),
                                     (0, 0)))
    xf = jnp.pad(x_foot, ((0, 0), (0, 128 - x_foot.shape[1])))
    Wf = jnp.pad(enc["foot"]["W"], ((0, 128 - enc["foot"]["W"].shape[0]),
                                    (0, 0)))

    h_b = _mm_act(x_base, enc["base"]["W"], enc["base"]["b"], True)
    h_j = _mm_act(xj, Wj, enc["joint"]["b"], True)
    h_f = _mm_act(xf, Wf, enc["foot"]["b"], True)

    src_gt, dst_gt = ei_gt[0], ei_gt[1]
    src_gs, dst_gs = ei_gs[0], ei_gs[1]
    src_gr, dst_gr = ei_gr[0], ei_gr[1]
    src_bj, dst_bj = ei_bj[0], ei_bj[1]
    src_jj, dst_jj = ei_jj[0], ei_jj[1]
    src_jf, dst_jf = ei_jf[0], ei_jf[1]
    E_bb = src_gt.shape[0]
    E_bj = src_bj.shape[0]
    E_jj = src_jj.shape[0]
    E_jf = src_jf.shape[0]

    seg_deg = _seg_sum(((NB, ((E_bb, 4096),)), (NB, ((E_bb, 4096),))), 8192)
    seg_layer = _seg_sum(
        ((NB, ((E_bb, NB),)), (NB, ((E_bb, NB),)), (NB, ((E_bb, NB),)),
         (NJ, ((E_bj, NB),)), (NJ, ((E_jj, NJ),)), (NF, ((E_jf, NJ),))),
        8192)

    ones_tab = jnp.ones((4096, H), jnp.float32)
    zsrc = jnp.zeros_like(src_gt)
    d_gt, d_gs = seg_deg(ones_tab, zsrc, dst_gt, ones_tab, zsrc, dst_gs)

    bt = params["bt"]
    dec = params["dec"]
    Wd = jnp.pad(dec["W"], ((0, 0), (0, 128 - dec["W"].shape[1])))
    bd = jnp.pad(dec["b"], ((0, 128 - dec["b"].shape[0]),))

    out = None
    for l in range(3):
        lp = params["convs"][l]
        gt, gs, gr = (lp["base_gt_base"], lp["base_gs_base"],
                      lp["base_gr_base"])
        bj, jj, jf = (lp["base_connect_joint"], lp["joint_connect_joint"],
                      lp["joint_connect_foot"])

        a_gt, a_gs, a_gr, a_bj, a_jj, a_f = seg_layer(
            h_b, src_gt, dst_gt, h_b, src_gs, dst_gs, h_b, src_gr, dst_gr,
            h_b, src_bj, dst_bj, h_j, src_jj, dst_jj, h_j, src_jf, dst_jf)

        wroot_b = gt["W_root"] + gs["W_root"] + gr["W_root"]
        bsum_b = gt["b_rel"] + gs["b_rel"] + gr["b_rel"]
        wroot_j = bj["W_root"] + jj["W_root"]
        bsum_j = bj["b_rel"] + jj["b_rel"]

        h_b = _combine_base(h_b, a_gt, a_gs, a_gr, d_gt, d_gs, gt["W_rel"],
                            gs["W_rel"], gr["W_rel"], wroot_b, bsum_b,
                            bt["W1"], bt["b1"], bt["W2"], bt["b2"])
        h_j = _combine_two(h_j, a_bj, a_jj, bj["W_rel"], jj["W_rel"],
                           wroot_j, bsum_j)
        if l < 2:
            h_f = _combine_two(h_f, a_f, None, jf["W_rel"], None,
                               jf["W_root"], jf["b_rel"])
        else:
            out = _combine_foot_dec(h_f, a_f, jf["W_rel"], jf["W_root"],
                                    jf["b_rel"], Wd, bd)
    return out[:, :1]
